# baseline (speedup 1.0000x reference)
import jax
import jax.numpy as jnp
from jax import lax
from jax.experimental import pallas as pl
from jax.experimental.pallas import tpu as pltpu

_B, _N, _E = 32, 64, 256
_D = 128
_DM = 64
_TEMP = 0.1
_GS_ITERS = 20
_NUM_PROP = 5
_PP = 16


def _dgT(pt, x):
    return lax.dot_general(pt, x, (((0,), (0,)), ((), ())),
                           preferred_element_type=jnp.float32)


def _isonet_body(nfq_ref, nfc_ref, eiq_ref, eic_ref,
                 encW_ref, encb_ref, W1cat_ref, b1_ref, rb1_ref,
                 mW2_ref, b2_ref, rW2_ref, rb2_ref,
                 Wih_ref, Whh_ref, bih_ref, bhh_ref,
                 lW1_ref, lb1_ref, lW2_ref, lb2_ref,
                 out_ref):
    f32 = jnp.float32
    encW = encW_ref[...]
    encb = encb_ref[...]
    W1cat = W1cat_ref[...]
    b1 = b1_ref[...]
    rb1 = rb1_ref[...]
    mW2 = mW2_ref[...]
    b2 = b2_ref[...]
    rW2 = rW2_ref[...]
    rb2 = rb2_ref[...]
    Wih = Wih_ref[...]
    Whh = Whh_ref[...]
    bih = bih_ref[...]
    bhh = bhh_ref[...]
    lW1 = lW1_ref[...]
    lb1 = lb1_ref[...]
    lW2 = lW2_ref[...]
    lb2 = lb2_ref[...]

    NN, EE = 2 * _N, 2 * _E
    rows = lax.broadcasted_iota(jnp.int32, (NN, EE), 0)
    PfT, PtT, nf = [], [], []
    for p in range(_PP):
        fi = jnp.concatenate(
            [eiq_ref[p, 0:1, :], eic_ref[p, 0:1, :] + _N], axis=1)
        ti = jnp.concatenate(
            [eiq_ref[p, 1:2, :], eic_ref[p, 1:2, :] + _N], axis=1)
        PfT.append((rows == fi).astype(f32))
        PtT.append((rows == ti).astype(f32))
        nf0 = jnp.concatenate([nfq_ref[p], nfc_ref[p]], axis=0)
        nf.append(jnp.dot(nf0, encW, preferred_element_type=f32) + encb)
    for _ in range(_NUM_PROP):
        A = [jnp.dot(nf[p], W1cat, preferred_element_type=f32)
             for p in range(_PP)]
        Gf = [_dgT(PfT[p], A[p][:, :256]) for p in range(_PP)]
        Gt = [_dgT(PtT[p], A[p][:, 256:]) for p in range(_PP)]
        hm = [jnp.maximum(Gf[p][:, :128] + Gt[p][:, :128] + b1, 0.0)
              for p in range(_PP)]
        hr = [jnp.maximum(Gf[p][:, 128:] + Gt[p][:, 128:] + rb1, 0.0)
              for p in range(_PP)]
        msg = [jnp.dot(hm[p], mW2, preferred_element_type=f32) + b2
               for p in range(_PP)]
        rmsg = [jnp.dot(hr[p], rW2, preferred_element_type=f32) + rb2
                for p in range(_PP)]
        agg = [jnp.dot(PtT[p], msg[p], preferred_element_type=f32)
               + jnp.dot(PfT[p], rmsg[p], preferred_element_type=f32)
               for p in range(_PP)]
        gi = [jnp.dot(agg[p], Wih, preferred_element_type=f32) + bih
              for p in range(_PP)]
        gh = [jnp.dot(nf[p], Whh, preferred_element_type=f32) + bhh
              for p in range(_PP)]
        nxt = []
        for p in range(_PP):
            r = jax.nn.sigmoid(gi[p][:, :_D] + gh[p][:, :_D])
            z = jax.nn.sigmoid(gi[p][:, _D:2 * _D] + gh[p][:, _D:2 * _D])
            n = jnp.tanh(gi[p][:, 2 * _D:] + r * gh[p][:, 2 * _D:])
            nxt.append((1.0 - z) * n + z * nf[p])
        nf = nxt
    parts = []
    for p in range(_PP):
        A = jnp.dot(nf[p], W1cat, preferred_element_type=f32)
        Gf = _dgT(PfT[p], A[:, 0:128])
        Gt = _dgT(PtT[p], A[:, 256:384])
        h = jnp.maximum(Gf + Gt + b1, 0.0)
        e_both = jnp.dot(h, mW2, preferred_element_type=f32) + b2
        l_both = jnp.dot(
            jnp.maximum(jnp.dot(e_both, lW1, preferred_element_type=f32) + lb1, 0.0),
            lW2, preferred_element_type=f32) + lb2
        sim = lax.dot_general(l_both[:_E], l_both[_E:],
                              (((1,), (1,)), ((), ())),
                              preferred_element_type=f32)
        la = sim * (1.0 / _TEMP)
        rm = jnp.max(la, axis=1, keepdims=True)
        M0 = jnp.exp(la - rm)
        parts.append((e_both[:_E], e_both[_E:], M0))

    M0 = jnp.concatenate([m[None] for (_, _, m) in parts], axis=0)
    u = 1.0 / jnp.sum(M0, axis=2, keepdims=True)
    v = 1.0 / jnp.sum(M0 * u, axis=1, keepdims=True)
    for _ in range(_GS_ITERS - 1):
        u = 1.0 / jnp.sum(M0 * v, axis=2, keepdims=True)
        v = 1.0 / jnp.sum(M0 * u, axis=1, keepdims=True)
    plan = M0 * u * v

    for p in range(_PP):
        eq, ec, _ = parts[p]
        pe = jnp.dot(plan[p], ec, preferred_element_type=f32)
        s = -jnp.sum(jnp.maximum(eq - pe, 0.0))
        out_ref[p] = jnp.broadcast_to(jnp.reshape(s, (1, 1)), (1, 128))


@jax.jit
def _run(nf_q, nf_c, ei_q, ei_c, enc_W, enc_b, W1cat, b1, rb1, mW2, b2,
         rW2, rb2, Wih, Whh, bih, bhh, lW1, lb1, lW2, lb2):
    full = lambda shape: pl.BlockSpec(shape, lambda b: (0,) * len(shape))
    out = pl.pallas_call(
        _isonet_body,
        grid=(_B // _PP,),
        in_specs=[
            pl.BlockSpec((_PP, _N, _D), lambda b: (b, 0, 0)),
            pl.BlockSpec((_PP, _N, _D), lambda b: (b, 0, 0)),
            pl.BlockSpec((_PP, 2, _E), lambda b: (b, 0, 0)),
            pl.BlockSpec((_PP, 2, _E), lambda b: (b, 0, 0)),
            full((_D, _D)), full((1, _D)),
            full((_D, 4 * _D)), full((1, _D)), full((1, _D)),
            full((_D, _DM)), full((1, _DM)),
            full((_D, _DM)), full((1, _DM)),
            full((_DM, 3 * _D)), full((_D, 3 * _D)),
            full((1, 3 * _D)), full((1, 3 * _D)),
            full((_DM, 16)), full((1, 16)), full((16, 16)), full((1, 16)),
        ],
        out_specs=pl.BlockSpec((_PP, 1, 128), lambda b: (b, 0, 0)),
        out_shape=jax.ShapeDtypeStruct((_B, 1, 128), jnp.float32),
        compiler_params=pltpu.CompilerParams(
            dimension_semantics=("parallel",),
        ),
    )(nf_q, nf_c, ei_q, ei_c, enc_W, enc_b, W1cat, b1, rb1, mW2, b2,
      rW2, rb2, Wih, Whh, bih, bhh, lW1, lb1, lW2, lb2)
    return out[:, 0, 0]


def kernel(node_features_q, node_features_c, edge_index_q, edge_index_c,
           enc_W, enc_b, msg_W1, msg_b1, msg_W2, msg_b2,
           rmsg_W1, rmsg_b1, rmsg_W2, rmsg_b2,
           gru_Wih, gru_Whh, gru_bih, gru_bhh,
           lrl_W1, lrl_b1, lrl_W2, lrl_b2):
    W1cat = jnp.concatenate(
        [msg_W1[:_D], rmsg_W1[_D:], msg_W1[_D:], rmsg_W1[:_D]], axis=1)
    r2 = lambda x: jnp.reshape(x, (1, -1))
    return _run(node_features_q, node_features_c,
                edge_index_q.astype(jnp.int32), edge_index_c.astype(jnp.int32),
                enc_W, r2(enc_b), W1cat, r2(msg_b1), r2(rmsg_b1),
                msg_W2, r2(msg_b2), rmsg_W2, r2(rmsg_b2),
                gru_Wih, gru_Whh, r2(gru_bih), r2(gru_bhh),
                lrl_W1, r2(lrl_b1), lrl_W2, r2(lrl_b2))


# speedup vs baseline: 1.3042x; 1.3042x over previous
import jax
import jax.numpy as jnp
from jax import lax
from jax.experimental import pallas as pl
from jax.experimental.pallas import tpu as pltpu

_B, _N, _E = 32, 64, 256
_D = 128
_DM = 64
_TEMP = 0.1
_GS_ITERS = 20
_NUM_PROP = 5
_PP = 8


def _dgT(pt, x):
    return lax.dot_general(pt, x, (((0,), (0,)), ((), ())),
                           preferred_element_type=jnp.float32)


def _isonet_body(nfq_ref, nfc_ref, eiq_ref, eic_ref,
                 encW_ref, encb_ref, W1cat_ref, b1_ref, rb1_ref,
                 mW2_ref, b2_ref, rW2_ref, rb2_ref,
                 Wih_ref, Whh_ref, bih_ref, bhh_ref,
                 lW1_ref, lb1_ref, lW2_ref, lb2_ref,
                 out_ref):
    f32 = jnp.float32
    encW = encW_ref[...]
    encb = encb_ref[...]
    W1cat = W1cat_ref[...]
    b1 = b1_ref[...]
    rb1 = rb1_ref[...]
    mW2 = mW2_ref[...]
    b2 = b2_ref[...]
    rW2 = rW2_ref[...]
    rb2 = rb2_ref[...]
    Wih = Wih_ref[...]
    Whh = Whh_ref[...]
    bih = bih_ref[...]
    bhh = bhh_ref[...]
    lW1 = lW1_ref[...]
    lb1 = lb1_ref[...]
    lW2 = lW2_ref[...]
    lb2 = lb2_ref[...]

    NN, EE = 2 * _N, 2 * _E
    rows = lax.broadcasted_iota(jnp.int32, (NN, EE), 0)
    PfT, PtT, nf = [], [], []
    for p in range(_PP):
        fi = jnp.concatenate(
            [eiq_ref[p, 0:1, :], eic_ref[p, 0:1, :] + _N], axis=1)
        ti = jnp.concatenate(
            [eiq_ref[p, 1:2, :], eic_ref[p, 1:2, :] + _N], axis=1)
        PfT.append((rows == fi).astype(jnp.bfloat16))
        PtT.append((rows == ti).astype(jnp.bfloat16))
        nf0 = jnp.concatenate([nfq_ref[p], nfc_ref[p]], axis=0)
        nf.append(jnp.dot(nf0, encW, preferred_element_type=f32) + encb)
    for _ in range(_NUM_PROP):
        A = [jnp.dot(nf[p], W1cat, preferred_element_type=f32)
             for p in range(_PP)]
        Gf = [_dgT(PfT[p], A[p][:, :256]) for p in range(_PP)]
        Gt = [_dgT(PtT[p], A[p][:, 256:]) for p in range(_PP)]
        hm = [jnp.maximum(Gf[p][:, :128] + Gt[p][:, :128] + b1, 0.0)
              for p in range(_PP)]
        hr = [jnp.maximum(Gf[p][:, 128:] + Gt[p][:, 128:] + rb1, 0.0)
              for p in range(_PP)]
        msg = [jnp.dot(hm[p], mW2, preferred_element_type=f32) + b2
               for p in range(_PP)]
        rmsg = [jnp.dot(hr[p], rW2, preferred_element_type=f32) + rb2
                for p in range(_PP)]
        agg = [jnp.dot(PtT[p], msg[p], preferred_element_type=f32)
               + jnp.dot(PfT[p], rmsg[p], preferred_element_type=f32)
               for p in range(_PP)]
        gi = [jnp.dot(agg[p], Wih, preferred_element_type=f32) + bih
              for p in range(_PP)]
        gh = [jnp.dot(nf[p], Whh, preferred_element_type=f32) + bhh
              for p in range(_PP)]
        nxt = []
        for p in range(_PP):
            r = jax.nn.sigmoid(gi[p][:, :_D] + gh[p][:, :_D])
            z = jax.nn.sigmoid(gi[p][:, _D:2 * _D] + gh[p][:, _D:2 * _D])
            n = jnp.tanh(gi[p][:, 2 * _D:] + r * gh[p][:, 2 * _D:])
            nxt.append((1.0 - z) * n + z * nf[p])
        nf = nxt
    parts = []
    for p in range(_PP):
        A = jnp.dot(nf[p], W1cat, preferred_element_type=f32)
        Gf = _dgT(PfT[p], A[:, 0:128])
        Gt = _dgT(PtT[p], A[:, 256:384])
        h = jnp.maximum(Gf + Gt + b1, 0.0)
        e_both = jnp.dot(h, mW2, preferred_element_type=f32) + b2
        l_both = jnp.dot(
            jnp.maximum(jnp.dot(e_both, lW1, preferred_element_type=f32) + lb1, 0.0),
            lW2, preferred_element_type=f32) + lb2
        sim = lax.dot_general(l_both[:_E], l_both[_E:],
                              (((1,), (1,)), ((), ())),
                              preferred_element_type=f32)
        la = sim * (1.0 / _TEMP)
        rm = jnp.max(la, axis=1, keepdims=True)
        M0 = jnp.exp(la - rm)
        parts.append((e_both[:_E], e_both[_E:], M0))

    M0 = jnp.concatenate([m[None] for (_, _, m) in parts], axis=0)
    u = 1.0 / jnp.sum(M0, axis=2, keepdims=True)
    v = 1.0 / jnp.sum(M0 * u, axis=1, keepdims=True)
    for _ in range(_GS_ITERS - 1):
        u = 1.0 / jnp.sum(M0 * v, axis=2, keepdims=True)
        v = 1.0 / jnp.sum(M0 * u, axis=1, keepdims=True)
    plan = M0 * u * v

    for p in range(_PP):
        eq, ec, _ = parts[p]
        pe = jnp.dot(plan[p], ec, preferred_element_type=f32)
        s = -jnp.sum(jnp.maximum(eq - pe, 0.0))
        out_ref[p] = jnp.broadcast_to(jnp.reshape(s, (1, 1)), (1, 128))


@jax.jit
def _run(nf_q, nf_c, ei_q, ei_c, enc_W, enc_b, W1cat, b1, rb1, mW2, b2,
         rW2, rb2, Wih, Whh, bih, bhh, lW1, lb1, lW2, lb2):
    full = lambda shape: pl.BlockSpec(shape, lambda b: (0,) * len(shape))
    out = pl.pallas_call(
        _isonet_body,
        grid=(_B // _PP,),
        in_specs=[
            pl.BlockSpec((_PP, _N, _D), lambda b: (b, 0, 0)),
            pl.BlockSpec((_PP, _N, _D), lambda b: (b, 0, 0)),
            pl.BlockSpec((_PP, 2, _E), lambda b: (b, 0, 0)),
            pl.BlockSpec((_PP, 2, _E), lambda b: (b, 0, 0)),
            full((_D, _D)), full((1, _D)),
            full((_D, 4 * _D)), full((1, _D)), full((1, _D)),
            full((_D, _DM)), full((1, _DM)),
            full((_D, _DM)), full((1, _DM)),
            full((_DM, 3 * _D)), full((_D, 3 * _D)),
            full((1, 3 * _D)), full((1, 3 * _D)),
            full((_DM, 16)), full((1, 16)), full((16, 16)), full((1, 16)),
        ],
        out_specs=pl.BlockSpec((_PP, 1, 128), lambda b: (b, 0, 0)),
        out_shape=jax.ShapeDtypeStruct((_B, 1, 128), jnp.float32),
        compiler_params=pltpu.CompilerParams(
            dimension_semantics=("parallel",),
        ),
    )(nf_q, nf_c, ei_q, ei_c, enc_W, enc_b, W1cat, b1, rb1, mW2, b2,
      rW2, rb2, Wih, Whh, bih, bhh, lW1, lb1, lW2, lb2)
    return out[:, 0, 0]


def kernel(node_features_q, node_features_c, edge_index_q, edge_index_c,
           enc_W, enc_b, msg_W1, msg_b1, msg_W2, msg_b2,
           rmsg_W1, rmsg_b1, rmsg_W2, rmsg_b2,
           gru_Wih, gru_Whh, gru_bih, gru_bhh,
           lrl_W1, lrl_b1, lrl_W2, lrl_b2):
    W1cat = jnp.concatenate(
        [msg_W1[:_D], rmsg_W1[_D:], msg_W1[_D:], rmsg_W1[:_D]], axis=1)
    r2 = lambda x: jnp.reshape(x, (1, -1))
    return _run(node_features_q, node_features_c,
                edge_index_q.astype(jnp.int32), edge_index_c.astype(jnp.int32),
                enc_W, r2(enc_b), W1cat, r2(msg_b1), r2(rmsg_b1),
                msg_W2, r2(msg_b2), rmsg_W2, r2(rmsg_b2),
                gru_Wih, gru_Whh, r2(gru_bih), r2(gru_bhh),
                lrl_W1, r2(lrl_b1), lrl_W2, r2(lrl_b2))


# speedup vs baseline: 1.4477x; 1.1100x over previous
import jax
import jax.numpy as jnp
from jax import lax
from jax.experimental import pallas as pl
from jax.experimental.pallas import tpu as pltpu

_B, _N, _E = 32, 64, 256
_D = 128
_DM = 64
_TEMP = 0.1
_GS_ITERS = 20
_NUM_PROP = 5
_PP = 8


def _dgT(pt, x):
    return lax.dot_general(pt, x, (((0,), (0,)), ((), ())),
                           preferred_element_type=jnp.float32)


def _isonet_body(nfq_ref, nfc_ref, eiq_ref, eic_ref,
                 encW_ref, encb_ref, W1cat_ref, b1_ref, rb1_ref,
                 mW2_ref, b2_ref, rW2_ref, rb2_ref,
                 Wih_ref, Whh_ref, bih_ref, bhh_ref,
                 lW1_ref, lb1_ref, lW2_ref, lb2_ref,
                 out_ref):
    f32 = jnp.float32
    encW = encW_ref[...]
    encb = encb_ref[...]
    W1cat = W1cat_ref[...]
    b1 = b1_ref[...]
    rb1 = rb1_ref[...]
    mW2 = mW2_ref[...]
    b2 = b2_ref[...]
    rW2 = rW2_ref[...]
    rb2 = rb2_ref[...]
    Wih = Wih_ref[...]
    Whh = Whh_ref[...]
    bih = bih_ref[...]
    bhh = bhh_ref[...]
    lW1 = lW1_ref[...]
    lb1 = lb1_ref[...]
    lW2 = lW2_ref[...]
    lb2 = lb2_ref[...]
    bcat = jnp.concatenate([b1, rb1], axis=1)

    NN, EE = 2 * _N, 2 * _E
    rows = lax.broadcasted_iota(jnp.int32, (NN, EE), 0)
    PT2, PT3, nf = [], [], []
    for p in range(_PP):
        fi = jnp.concatenate(
            [eiq_ref[p, 0:1, :], eic_ref[p, 0:1, :] + _N], axis=1)
        ti = jnp.concatenate(
            [eiq_ref[p, 1:2, :], eic_ref[p, 1:2, :] + _N], axis=1)
        PfT = (rows == fi).astype(jnp.bfloat16)
        PtT = (rows == ti).astype(jnp.bfloat16)
        PT2.append(jnp.concatenate([PfT, PtT], axis=0))
        PT3.append(jnp.concatenate([PtT, PfT], axis=1))
        nf0 = jnp.concatenate([nfq_ref[p], nfc_ref[p]], axis=0)
        nf.append(jnp.dot(nf0, encW, preferred_element_type=f32) + encb)
    for _ in range(_NUM_PROP):
        A = [jnp.dot(nf[p], W1cat, preferred_element_type=f32)
             for p in range(_PP)]
        A2 = [jnp.concatenate([A[p][:, :256], A[p][:, 256:]], axis=0)
              for p in range(_PP)]
        G = [_dgT(PT2[p], A2[p]) for p in range(_PP)]
        h = [jnp.maximum(G[p] + bcat, 0.0) for p in range(_PP)]
        msg2 = [jnp.concatenate(
                    [jnp.dot(h[p][:, :128], mW2, preferred_element_type=f32) + b2,
                     jnp.dot(h[p][:, 128:], rW2, preferred_element_type=f32) + rb2],
                    axis=0)
                for p in range(_PP)]
        agg = [jnp.dot(PT3[p], msg2[p], preferred_element_type=f32)
               for p in range(_PP)]
        gi = [jnp.dot(agg[p], Wih, preferred_element_type=f32) + bih
              for p in range(_PP)]
        gh = [jnp.dot(nf[p], Whh, preferred_element_type=f32) + bhh
              for p in range(_PP)]
        nxt = []
        for p in range(_PP):
            r = jax.nn.sigmoid(gi[p][:, :_D] + gh[p][:, :_D])
            z = jax.nn.sigmoid(gi[p][:, _D:2 * _D] + gh[p][:, _D:2 * _D])
            n = jnp.tanh(gi[p][:, 2 * _D:] + r * gh[p][:, 2 * _D:])
            nxt.append((1.0 - z) * n + z * nf[p])
        nf = nxt
    parts = []
    for p in range(_PP):
        A = jnp.dot(nf[p], W1cat, preferred_element_type=f32)
        A2f = jnp.concatenate([A[:, 0:128], A[:, 256:384]], axis=0)
        G = _dgT(PT2[p], A2f)
        h = jnp.maximum(G + b1, 0.0)
        e_both = jnp.dot(h, mW2, preferred_element_type=f32) + b2
        l_both = jnp.dot(
            jnp.maximum(jnp.dot(e_both, lW1, preferred_element_type=f32) + lb1, 0.0),
            lW2, preferred_element_type=f32) + lb2
        sim = lax.dot_general(l_both[:_E], l_both[_E:],
                              (((1,), (1,)), ((), ())),
                              preferred_element_type=f32)
        la = sim * (1.0 / _TEMP)
        rm = jnp.max(la, axis=1, keepdims=True)
        M0 = jnp.exp(la - rm)
        parts.append((e_both[:_E], e_both[_E:], M0))

    M0 = jnp.concatenate([m[None] for (_, _, m) in parts], axis=0)
    u = 1.0 / jnp.sum(M0, axis=2, keepdims=True)
    v = 1.0 / jnp.sum(M0 * u, axis=1, keepdims=True)
    for _ in range(_GS_ITERS - 1):
        u = 1.0 / jnp.sum(M0 * v, axis=2, keepdims=True)
        v = 1.0 / jnp.sum(M0 * u, axis=1, keepdims=True)
    plan = M0 * u * v

    for p in range(_PP):
        eq, ec, _ = parts[p]
        pe = jnp.dot(plan[p], ec, preferred_element_type=f32)
        s = -jnp.sum(jnp.maximum(eq - pe, 0.0))
        out_ref[p] = jnp.broadcast_to(jnp.reshape(s, (1, 1)), (1, 128))


@jax.jit
def _run(nf_q, nf_c, ei_q, ei_c, enc_W, enc_b, W1cat, b1, rb1, mW2, b2,
         rW2, rb2, Wih, Whh, bih, bhh, lW1, lb1, lW2, lb2):
    full = lambda shape: pl.BlockSpec(shape, lambda b: (0,) * len(shape))
    out = pl.pallas_call(
        _isonet_body,
        grid=(_B // _PP,),
        in_specs=[
            pl.BlockSpec((_PP, _N, _D), lambda b: (b, 0, 0)),
            pl.BlockSpec((_PP, _N, _D), lambda b: (b, 0, 0)),
            pl.BlockSpec((_PP, 2, _E), lambda b: (b, 0, 0)),
            pl.BlockSpec((_PP, 2, _E), lambda b: (b, 0, 0)),
            full((_D, _D)), full((1, _D)),
            full((_D, 4 * _D)), full((1, _D)), full((1, _D)),
            full((_D, _DM)), full((1, _DM)),
            full((_D, _DM)), full((1, _DM)),
            full((_DM, 3 * _D)), full((_D, 3 * _D)),
            full((1, 3 * _D)), full((1, 3 * _D)),
            full((_DM, 16)), full((1, 16)), full((16, 16)), full((1, 16)),
        ],
        out_specs=pl.BlockSpec((_PP, 1, 128), lambda b: (b, 0, 0)),
        out_shape=jax.ShapeDtypeStruct((_B, 1, 128), jnp.float32),
        compiler_params=pltpu.CompilerParams(
            dimension_semantics=("parallel",),
        ),
    )(nf_q, nf_c, ei_q, ei_c, enc_W, enc_b, W1cat, b1, rb1, mW2, b2,
      rW2, rb2, Wih, Whh, bih, bhh, lW1, lb1, lW2, lb2)
    return out[:, 0, 0]


def kernel(node_features_q, node_features_c, edge_index_q, edge_index_c,
           enc_W, enc_b, msg_W1, msg_b1, msg_W2, msg_b2,
           rmsg_W1, rmsg_b1, rmsg_W2, rmsg_b2,
           gru_Wih, gru_Whh, gru_bih, gru_bhh,
           lrl_W1, lrl_b1, lrl_W2, lrl_b2):
    W1cat = jnp.concatenate(
        [msg_W1[:_D], rmsg_W1[_D:], msg_W1[_D:], rmsg_W1[:_D]], axis=1)
    r2 = lambda x: jnp.reshape(x, (1, -1))
    return _run(node_features_q, node_features_c,
                edge_index_q.astype(jnp.int32), edge_index_c.astype(jnp.int32),
                enc_W, r2(enc_b), W1cat, r2(msg_b1), r2(rmsg_b1),
                msg_W2, r2(msg_b2), rmsg_W2, r2(rmsg_b2),
                gru_Wih, gru_Whh, r2(gru_bih), r2(gru_bhh),
                lrl_W1, r2(lrl_b1), lrl_W2, r2(lrl_b2))
